# baseline (speedup 1.0000x reference)
import functools
import math

import jax
import jax.numpy as jnp
from jax import lax
from jax.experimental import pallas as pl
from jax.experimental.pallas import tpu as pltpu
from jax.experimental.pallas import tpu_sc as plsc

_LANES = 16
_CHUNK = 128


def _positional_encoding(seq_len, d_model):
    position = jnp.arange(seq_len, dtype=jnp.float32)[:, None]
    div_term = jnp.exp(
        jnp.arange(0, d_model, 2, dtype=jnp.float32)
        * (-math.log(10000.0) / d_model)
    )
    pe = jnp.zeros((seq_len, d_model), dtype=jnp.float32)
    pe = pe.at[:, 0::2].set(jnp.sin(position * div_term))
    pe = pe.at[:, 1::2].set(jnp.cos(position * div_term))
    return pe


@functools.lru_cache(maxsize=None)
def _make_lookup(n_rows, seq_len, d_model):
    info = plsc.get_sparse_core_info()
    nc, ns = info.num_cores, info.num_subcores
    nw = nc * ns
    assert n_rows % (nw * seq_len) == 0
    rows_per_w = n_rows // nw
    assert rows_per_w % _CHUNK == 0
    n_chunks = rows_per_w // _CHUNK
    assert _CHUNK < seq_len
    mesh = plsc.VectorSubcoreMesh(core_axis_name="c", subcore_axis_name="s")

    @functools.partial(
        pl.kernel,
        mesh=mesh,
        out_type=jax.ShapeDtypeStruct((n_rows, d_model), jnp.float32),
        scratch_types=[
            pltpu.VMEM((seq_len, d_model), jnp.float32),
            pltpu.VMEM((_CHUNK,), jnp.int32),
            pltpu.VMEM((_CHUNK, d_model), jnp.float32),
            pltpu.SemaphoreType.DMA,
        ],
        compiler_params=pltpu.CompilerParams(use_tc_tiling_on_sc=False),
    )
    def lookup(idx_hbm, table_hbm, pe_hbm, out_hbm, pe_v, idx_v, rows_v, sem):
        wid = lax.axis_index("s") * nc + lax.axis_index("c")
        base = wid * rows_per_w
        pltpu.sync_copy(pe_hbm, pe_v)

        def chunk_body(c, carry):
            off = base + c * _CHUNK
            pe_off = lax.rem(c * _CHUNK, seq_len)
            pltpu.sync_copy(idx_hbm.at[pl.ds(off, _CHUNK)], idx_v)
            pltpu.async_copy(table_hbm.at[idx_v], rows_v, sem).wait()

            def row_body(i, carry2):
                p = pe_off + i
                p = lax.select(p >= seq_len, p - seq_len, p)
                for k in range(d_model // _LANES):
                    sl = pl.ds(k * _LANES, _LANES)
                    plsc.addupdate(rows_v.at[i, sl], pe_v[p, sl])
                return carry2

            lax.fori_loop(0, _CHUNK, row_body, 0, unroll=2)
            pltpu.sync_copy(rows_v, out_hbm.at[pl.ds(off, _CHUNK)])
            return carry

        lax.fori_loop(0, n_chunks, chunk_body, 0)

    return lookup


def kernel(x, table):
    b, l = x.shape
    d = table.shape[1]
    pe = _positional_encoding(l, d)
    flat_idx = x.reshape(-1).astype(jnp.int32)
    lookup = _make_lookup(b * l, l, d)
    out = lookup(flat_idx, table, pe)
    return out.reshape(b, l, d)


# speedup vs baseline: 1.0813x; 1.0813x over previous
import functools
import math

import jax
import jax.numpy as jnp
from jax import lax
from jax.experimental import pallas as pl
from jax.experimental.pallas import tpu as pltpu
from jax.experimental.pallas import tpu_sc as plsc

_LANES = 16
_CHUNK = 256


def _positional_encoding(seq_len, d_model):
    position = jnp.arange(seq_len, dtype=jnp.float32)[:, None]
    div_term = jnp.exp(
        jnp.arange(0, d_model, 2, dtype=jnp.float32)
        * (-math.log(10000.0) / d_model)
    )
    pe = jnp.zeros((seq_len, d_model), dtype=jnp.float32)
    pe = pe.at[:, 0::2].set(jnp.sin(position * div_term))
    pe = pe.at[:, 1::2].set(jnp.cos(position * div_term))
    return pe


@functools.lru_cache(maxsize=None)
def _make_gather(vocab, seq_len, batch, d_model):
    info = plsc.get_sparse_core_info()
    nc, ns = info.num_cores, info.num_subcores
    nw = nc * ns
    n_rows = seq_len * batch
    n_items = n_rows // _CHUNK
    assert n_items % nw == 0
    items_per_w = n_items // nw
    mesh = plsc.VectorSubcoreMesh(core_axis_name="c", subcore_axis_name="s")

    @functools.partial(
        pl.kernel,
        mesh=mesh,
        out_type=jax.ShapeDtypeStruct((n_rows, d_model), jnp.float32),
        scratch_types=[
            pltpu.VMEM((2, 128), jnp.int32),
            pltpu.VMEM((_CHUNK, 128), jnp.float32),
            pltpu.VMEM((_CHUNK, 64), jnp.float32),
            pltpu.VMEM((seq_len, 128), jnp.float32),
            pltpu.SemaphoreType.DMA,
        ],
        compiler_params=pltpu.CompilerParams(use_tc_tiling_on_sc=True),
    )
    def gather_k(idx_hbm, tab_hbm, pe_hbm, out_hbm, idx_v, rows_v, acc_v, pe_v, sem):
        wid = lax.axis_index("s") * nc + lax.axis_index("c")
        pltpu.sync_copy(pe_hbm, pe_v)

        def item_body(k, carry):
            off = (k * nw + wid) * _CHUNK
            p0 = lax.rem(off, seq_len)
            for q in range(_CHUNK // 128):
                pltpu.sync_copy(
                    idx_hbm.at[pl.ds(off + q * 128, 128)], idx_v.at[q]
                )
            cps = [
                pltpu.async_copy(
                    tab_hbm.at[idx_v.at[q]],
                    rows_v.at[pl.ds(q * 128, 128)],
                    sem,
                )
                for q in range(_CHUNK // 128)
            ]
            for cp in cps:
                cp.wait()

            def row_body(i, c2):
                p = p0 + i
                p = lax.select(p >= seq_len, p - seq_len, p)
                p = lax.select(p >= seq_len, p - seq_len, p)
                for j in range(4):
                    sl = pl.ds(_LANES * j, _LANES)
                    acc_v[i, sl] = rows_v[i, sl] + pe_v[p, sl]
                return c2

            lax.fori_loop(0, _CHUNK, row_body, 0)
            pltpu.sync_copy(acc_v, out_hbm.at[pl.ds(off, _CHUNK)])
            return carry

        lax.fori_loop(0, items_per_w, item_body, 0)

    return gather_k


def kernel(x, table):
    b, l = x.shape
    v, d = table.shape
    flat_idx = x.reshape(-1).astype(jnp.int32)
    tpad = jnp.pad(table, ((0, 0), (0, 128 - d)))
    pe = _positional_encoding(l, d)
    pe_pad = jnp.pad(pe, ((0, 0), (0, 128 - d)))
    gather_k = _make_gather(v, l, b, d)
    out = gather_k(flat_idx, tpad, pe_pad)
    return out.reshape(b, l, d)


# speedup vs baseline: 1.2020x; 1.1117x over previous
import functools
import math

import jax
import jax.numpy as jnp
from jax import lax
from jax.experimental import pallas as pl
from jax.experimental.pallas import tpu as pltpu
from jax.experimental.pallas import tpu_sc as plsc

_LANES = 16
_CHUNK = 128


def _positional_encoding(seq_len, d_model):
    position = jnp.arange(seq_len, dtype=jnp.float32)[:, None]
    div_term = jnp.exp(
        jnp.arange(0, d_model, 2, dtype=jnp.float32)
        * (-math.log(10000.0) / d_model)
    )
    pe = jnp.zeros((seq_len, d_model), dtype=jnp.float32)
    pe = pe.at[:, 0::2].set(jnp.sin(position * div_term))
    pe = pe.at[:, 1::2].set(jnp.cos(position * div_term))
    return pe


@functools.lru_cache(maxsize=None)
def _make_gather(vocab, seq_len, batch, d_model):
    info = plsc.get_sparse_core_info()
    nc, ns = info.num_cores, info.num_subcores
    nw = nc * ns
    n_rows = seq_len * batch
    n_items = n_rows // _CHUNK
    assert n_items % nw == 0
    items_per_w = n_items // nw
    assert items_per_w % 2 == 0
    assert _CHUNK < seq_len
    mesh = plsc.VectorSubcoreMesh(core_axis_name="c", subcore_axis_name="s")

    @functools.partial(
        pl.kernel,
        mesh=mesh,
        out_type=jax.ShapeDtypeStruct((n_rows, d_model), jnp.float32),
        scratch_types=[
            pltpu.VMEM((2, _CHUNK), jnp.int32),
            pltpu.VMEM((_CHUNK, 128), jnp.float32),
            pltpu.VMEM((_CHUNK, 128), jnp.float32),
            pltpu.VMEM((_CHUNK, d_model), jnp.float32),
            pltpu.VMEM((_CHUNK, d_model), jnp.float32),
            pltpu.VMEM((seq_len, 128), jnp.float32),
            pltpu.SemaphoreType.DMA,
            pltpu.SemaphoreType.DMA,
            pltpu.SemaphoreType.DMA,
            pltpu.SemaphoreType.DMA,
        ],
        compiler_params=pltpu.CompilerParams(use_tc_tiling_on_sc=True),
    )
    def gather_k(
        idx_hbm, tab_hbm, pe_hbm, out_hbm,
        idx_v, rows0, rows1, acc0, acc1, pe_v,
        gsem0, gsem1, wsem0, wsem1,
    ):
        rows = (rows0, rows1)
        acc = (acc0, acc1)
        gsem = (gsem0, gsem1)
        wsem = (wsem0, wsem1)
        wid = lax.axis_index("s") * nc + lax.axis_index("c")
        base = wid * items_per_w
        pltpu.sync_copy(pe_hbm, pe_v)

        def fire(item, buf):
            off = (base + item) * _CHUNK
            pltpu.sync_copy(idx_hbm.at[pl.ds(off, _CHUNK)], idx_v.at[buf])
            pltpu.async_copy(tab_hbm.at[idx_v.at[buf]], rows[buf], gsem[buf])

        def drain_gather(buf):
            pltpu.make_async_copy(
                tab_hbm.at[pl.ds(0, _CHUNK)], rows[buf], gsem[buf]
            ).wait()

        def drain_write(buf):
            pltpu.make_async_copy(
                acc[buf], out_hbm.at[pl.ds(0, _CHUNK)], wsem[buf]
            ).wait()

        fire(0, 0)

        def pair_body(k2, carry):
            for b2 in (0, 1):
                k = k2 * 2 + b2
                nxt = (b2 + 1) % 2
                if b2 == 0:
                    fire(k + 1, nxt)
                else:

                    @pl.when(k2 < items_per_w // 2 - 1)
                    def _():
                        fire(k + 1, nxt)

                drain_gather(b2)
                off = (base + k) * _CHUNK
                p0 = lax.rem(off, seq_len)

                def row_body(i, c2):
                    p = p0 + i
                    p = lax.select(p >= seq_len, p - seq_len, p)
                    for j in range(d_model // _LANES):
                        sl = pl.ds(_LANES * j, _LANES)
                        acc[b2][i, sl] = rows[b2][i, sl] + pe_v[p, sl]
                    return c2

                lax.fori_loop(0, _CHUNK, row_body, 0, unroll=2)

                @pl.when(k2 > 0)
                def _():
                    drain_write(b2)

                pltpu.async_copy(
                    acc[b2], out_hbm.at[pl.ds(off, _CHUNK)], wsem[b2]
                )
            return carry

        lax.fori_loop(0, items_per_w // 2, pair_body, 0)
        drain_write(0)
        drain_write(1)

    return gather_k


def kernel(x, table):
    b, l = x.shape
    v, d = table.shape
    flat_idx = x.reshape(-1).astype(jnp.int32)
    tpad = jnp.pad(table, ((0, 0), (0, 128 - d)))
    pe = _positional_encoding(l, d)
    pe_pad = jnp.pad(pe, ((0, 0), (0, 128 - d)))
    gather_k = _make_gather(v, l, b, d)
    out = gather_k(flat_idx, tpad, pe_pad)
    return out.reshape(b, l, d)


# speedup vs baseline: 1.2535x; 1.0428x over previous
import functools
import math

import jax
import jax.numpy as jnp
from jax import lax
from jax.experimental import pallas as pl
from jax.experimental.pallas import tpu as pltpu
from jax.experimental.pallas import tpu_sc as plsc

_LANES = 16
_CHUNK = 128


def _positional_encoding(seq_len, d_model):
    position = jnp.arange(seq_len, dtype=jnp.float32)[:, None]
    div_term = jnp.exp(
        jnp.arange(0, d_model, 2, dtype=jnp.float32)
        * (-math.log(10000.0) / d_model)
    )
    pe = jnp.zeros((seq_len, d_model), dtype=jnp.float32)
    pe = pe.at[:, 0::2].set(jnp.sin(position * div_term))
    pe = pe.at[:, 1::2].set(jnp.cos(position * div_term))
    return pe


@functools.lru_cache(maxsize=None)
def _make_gather(vocab, seq_len, batch, d_model):
    info = plsc.get_sparse_core_info()
    nc, ns = info.num_cores, info.num_subcores
    nw = nc * ns
    n_rows = seq_len * batch
    n_items = n_rows // _CHUNK
    assert n_items % nw == 0
    items_per_w = n_items // nw
    assert items_per_w % 2 == 0
    assert _CHUNK < seq_len
    mesh = plsc.VectorSubcoreMesh(core_axis_name="c", subcore_axis_name="s")

    @functools.partial(
        pl.kernel,
        mesh=mesh,
        out_type=jax.ShapeDtypeStruct((n_rows, d_model), jnp.float32),
        scratch_types=[
            pltpu.VMEM((items_per_w, _CHUNK), jnp.int32),
            pltpu.VMEM((_CHUNK, 128), jnp.float32),
            pltpu.VMEM((_CHUNK, 128), jnp.float32),
            pltpu.VMEM((_CHUNK, d_model), jnp.float32),
            pltpu.VMEM((_CHUNK, d_model), jnp.float32),
            pltpu.VMEM((seq_len, 128), jnp.float32),
            pltpu.SemaphoreType.DMA,
            pltpu.SemaphoreType.DMA,
            pltpu.SemaphoreType.DMA,
            pltpu.SemaphoreType.DMA,
        ],
        compiler_params=pltpu.CompilerParams(use_tc_tiling_on_sc=True),
    )
    def gather_k(
        idx_hbm, tab_hbm, pe_hbm, out_hbm,
        idx_v, rows0, rows1, acc0, acc1, pe_v,
        gsem0, gsem1, wsem0, wsem1,
    ):
        rows = (rows0, rows1)
        acc = (acc0, acc1)
        gsem = (gsem0, gsem1)
        wsem = (wsem0, wsem1)
        wid = lax.axis_index("s") * nc + lax.axis_index("c")
        base = wid * items_per_w
        pltpu.sync_copy(pe_hbm, pe_v)
        pltpu.sync_copy(idx_hbm.at[wid], idx_v)

        def fire(item, buf):
            pltpu.async_copy(tab_hbm.at[idx_v.at[item]], rows[buf], gsem[buf])

        def drain_gather(buf):
            pltpu.make_async_copy(
                tab_hbm.at[pl.ds(0, _CHUNK)], rows[buf], gsem[buf]
            ).wait()

        def drain_write(buf):
            pltpu.make_async_copy(
                acc[buf], out_hbm.at[pl.ds(0, _CHUNK)], wsem[buf]
            ).wait()

        fire(0, 0)

        def pair_body(k2, carry):
            for b2 in (0, 1):
                k = k2 * 2 + b2
                nxt = (b2 + 1) % 2
                if b2 == 0:
                    fire(k + 1, nxt)
                else:

                    @pl.when(k2 < items_per_w // 2 - 1)
                    def _():
                        fire(k + 1, nxt)

                drain_gather(b2)
                off = (base + k) * _CHUNK
                p0 = lax.rem(off, seq_len)

                def row_body(i, c2):
                    p = p0 + i
                    p = lax.select(p >= seq_len, p - seq_len, p)
                    for j in range(d_model // _LANES):
                        sl = pl.ds(_LANES * j, _LANES)
                        acc[b2][i, sl] = rows[b2][i, sl] + pe_v[p, sl]
                    return c2

                lax.fori_loop(0, _CHUNK, row_body, 0, unroll=4)

                @pl.when(k2 > 0)
                def _():
                    drain_write(b2)

                pltpu.async_copy(
                    acc[b2], out_hbm.at[pl.ds(off, _CHUNK)], wsem[b2]
                )
            return carry

        lax.fori_loop(0, items_per_w // 2, pair_body, 0)
        drain_write(0)
        drain_write(1)

    return gather_k


def kernel(x, table):
    b, l = x.shape
    v, d = table.shape
    info = plsc.get_sparse_core_info()
    nw = info.num_cores * info.num_subcores
    items_per_w = b * l // (_CHUNK * nw)
    idx3d = x.reshape(nw, items_per_w, _CHUNK).astype(jnp.int32)
    tpad = jnp.pad(table, ((0, 0), (0, 128 - d)))
    pe = _positional_encoding(l, d)
    pe_pad = jnp.pad(pe, ((0, 0), (0, 128 - d)))
    gather_k = _make_gather(v, l, b, d)
    out = gather_k(idx3d, tpad, pe_pad)
    return out.reshape(b, l, d)
